# baseline (speedup 1.0000x reference)
import functools

import jax
import jax.numpy as jnp
from jax import lax
from jax.experimental import pallas as pl
from jax.experimental.pallas import tpu as pltpu
from jax.experimental.pallas import tpu_sc as plsc

_T, _B, _C = 576, 64, 768
_RATIO = 0.75
_REMAIN = int(_T * (1 - _RATIO))
_MASKED = _T - _REMAIN
_NC, _NS = 2, 16
_NW = _NC * _NS
_CHUNK = 48
_MC = (_MASKED * _B) // (_NW * _CHUNK)
_DEPTH = 3


def _indexes():
    keys = jax.random.split(jax.random.key(42), _B)
    fwd = jax.vmap(lambda k: jax.random.permutation(k, _T))(keys).T
    bwd = jnp.argsort(fwd, axis=0)
    col = jnp.arange(_B, dtype=jnp.int32)[None, :]
    flat = fwd.astype(jnp.int32) * _B + col
    idx_m = flat[_REMAIN:].reshape(_NW, _MC, _CHUNK)
    onehot = (fwd[:_REMAIN].T[:, :, None]
              == jnp.arange(_T, dtype=jnp.int32)[None, None, :]).astype(jnp.float32)
    return fwd, bwd, idx_m, onehot


def _sc_gather_mask(mask_f, idx_m):
    mesh = plsc.VectorSubcoreMesh(core_axis_name="c", subcore_axis_name="s")

    @functools.partial(
        pl.kernel,
        mesh=mesh,
        out_type=jax.ShapeDtypeStruct((_MASKED * _B, _C), jnp.float32),
        scratch_types=(
            [pltpu.VMEM((_MC, _CHUNK), jnp.int32)]
            + [pltpu.VMEM((_CHUNK, _C), jnp.float32) for _ in range(_DEPTH)]
            + [pltpu.SemaphoreType.DMA for _ in range(2 * _DEPTH)]
        ),
    )
    def k(m_hbm, im_hbm, om_hbm, imv, *scratch):
        bufs = list(scratch[:_DEPTH])
        gsems = list(scratch[_DEPTH : 2 * _DEPTH])
        osems = list(scratch[2 * _DEPTH :])
        wid = lax.axis_index("s") * _NC + lax.axis_index("c")
        pltpu.sync_copy(im_hbm.at[wid], imv)
        n = _MC
        gh = [None] * n
        oh = [None] * n

        def out_copy(j):
            return pltpu.async_copy(
                bufs[j % _DEPTH],
                om_hbm.at[pl.ds((wid * _MC + j) * _CHUNK, _CHUNK)],
                osems[j % _DEPTH],
            )

        for i in range(n):
            if i >= _DEPTH:
                oh[i - _DEPTH].wait()
            gh[i] = pltpu.async_copy(m_hbm.at[imv.at[i]], bufs[i % _DEPTH], gsems[i % _DEPTH])
            if i >= 1:
                gh[i - 1].wait()
                oh[i - 1] = out_copy(i - 1)
        gh[n - 1].wait()
        oh[n - 1] = out_copy(n - 1)
        for j in range(max(0, n - _DEPTH), n):
            oh[j].wait()

    return k(mask_f, idx_m)


def _tc_onehot_patches(patches2, onehot):
    def body(oh_ref, x_ref, o_ref):
        a = oh_ref[0]
        x = x_ref[...]
        o_ref[...] = jax.lax.dot(
            a, x, precision=jax.lax.Precision.HIGHEST,
            preferred_element_type=jnp.float32,
        )

    return pl.pallas_call(
        body,
        grid=(_B,),
        in_specs=[
            pl.BlockSpec((1, _REMAIN, _T), lambda b: (b, 0, 0)),
            pl.BlockSpec((_T, _C), lambda b: (0, b)),
        ],
        out_specs=pl.BlockSpec((_REMAIN, _C), lambda b: (0, b)),
        out_shape=jax.ShapeDtypeStruct((_REMAIN, _B * _C), jnp.float32),
        compiler_params=pltpu.CompilerParams(
            dimension_semantics=("arbitrary",),
        ),
    )(onehot, patches2)


def kernel(patches, mask_patches):
    fwd, bwd, idx_m, onehot = _indexes()
    out_m = _sc_gather_mask(mask_patches.reshape(_T * _B, _C), idx_m)
    out_p2 = _tc_onehot_patches(patches.reshape(_T, _B * _C), onehot)
    return (
        out_p2.reshape(_REMAIN, _B, _C),
        out_m.reshape(_MASKED, _B, _C),
        fwd,
        bwd,
    )
